# Initial kernel scaffold; baseline (speedup 1.0000x reference)
import jax
import jax.numpy as jnp
from jax.experimental import pallas as pl


def kernel(t, sigma1, x_pred, x, segment_ids, beta1, one_hot_x, p_0, K):
    raise NotImplementedError("write your pallas kernel here")


# speedup vs baseline: 1.5627x; 1.5627x over previous
import functools

import jax
import jax.numpy as jnp
from jax import lax
from jax.experimental import pallas as pl
from jax.experimental.pallas import tpu as pltpu
from jax.experimental.pallas import tpu_sc as plsc

N = 100000
NUM_SEG = 512
D = 3
KDIM = 16
NC = 2
NS = 16
NW = NC * NS
CHUNK = 3136
LAST_CHUNK = N - 31 * CHUNK
NVEC = CHUNK // 16
LAST_NVEC = LAST_CHUNK // 16
BINS_PER_W = NUM_SEG // NS


def _sc_body(t_hbm, ids_hbm, xp_hbm, x_hbm, oh_hbm, p0_hbm, par_hbm,
             out_hbm,
             t_v, ids_v, xp_v, x_v, oh_v, p0_v, par_v,
             acc_c, acc_d, acc_n, res0, res1, res2, shared, rbuf):
    c = lax.axis_index("c")
    s = lax.axis_index("s")
    wid = c * NS + s
    base = wid * CHUNK

    pltpu.sync_copy(par_hbm, par_v)

    @pl.when(wid < NW - 1)
    def _stage_full():
        pltpu.sync_copy(t_hbm.at[pl.ds(base, CHUNK)], t_v)
        pltpu.sync_copy(ids_hbm.at[pl.ds(base, CHUNK)], ids_v)
        pltpu.sync_copy(xp_hbm.at[pl.ds(base * D, CHUNK * D)], xp_v)
        pltpu.sync_copy(x_hbm.at[pl.ds(base * D, CHUNK * D)], x_v)
        pltpu.sync_copy(oh_hbm.at[pl.ds(base * KDIM, CHUNK * KDIM)], oh_v)
        pltpu.sync_copy(p0_hbm.at[pl.ds(base * KDIM, CHUNK * KDIM)], p0_v)

    @pl.when(wid == NW - 1)
    def _stage_tail():
        pltpu.sync_copy(t_hbm.at[pl.ds(base, LAST_CHUNK)],
                        t_v.at[pl.ds(0, LAST_CHUNK)])
        pltpu.sync_copy(ids_hbm.at[pl.ds(base, LAST_CHUNK)],
                        ids_v.at[pl.ds(0, LAST_CHUNK)])
        pltpu.sync_copy(xp_hbm.at[pl.ds(base * D, LAST_CHUNK * D)],
                        xp_v.at[pl.ds(0, LAST_CHUNK * D)])
        pltpu.sync_copy(x_hbm.at[pl.ds(base * D, LAST_CHUNK * D)],
                        x_v.at[pl.ds(0, LAST_CHUNK * D)])
        pltpu.sync_copy(oh_hbm.at[pl.ds(base * KDIM, LAST_CHUNK * KDIM)],
                        oh_v.at[pl.ds(0, LAST_CHUNK * KDIM)])
        pltpu.sync_copy(p0_hbm.at[pl.ds(base * KDIM, LAST_CHUNK * KDIM)],
                        p0_v.at[pl.ds(0, LAST_CHUNK * KDIM)])

    zeros16 = jnp.zeros((16,), jnp.float32)
    for h in range(NUM_SEG // 16):
        acc_c[pl.ds(h * 16, 16)] = zeros16
        acc_d[pl.ds(h * 16, 16)] = zeros16
        acc_n[pl.ds(h * 16, 16)] = zeros16

    av = par_v[pl.ds(0, 16)]
    c1v = par_v[pl.ds(16, 16)]
    c2v = par_v[pl.ds(32, 16)]
    ones16 = jnp.full((16,), 1.0, jnp.float32)
    lanes = lax.iota(jnp.int32, 16)

    nvec = jnp.where(wid == NW - 1, LAST_NVEC, NVEC)

    def step(j, carry):
        node = j * 16 + lanes
        tv = t_v[pl.ds(j * 16, 16)]
        ids = ids_v[pl.ds(j * 16, 16)]
        w = jnp.exp(av * tv)
        se = zeros16
        for d in range(D):
            i3 = node * D + d
            dx = plsc.load_gather(xp_v, [i3]) - plsc.load_gather(x_v, [i3])
            se = se + dx * dx
        lc = c1v * w * se
        se2 = zeros16
        for k in range(KDIM):
            i16 = node * KDIM + k
            dq = plsc.load_gather(oh_v, [i16]) - plsc.load_gather(p0_v, [i16])
            se2 = se2 + dq * dq
        ld = c2v * tv * se2
        plsc.addupdate_scatter(acc_c, [ids], lc)
        plsc.addupdate_scatter(acc_d, [ids], ld)
        plsc.addupdate_scatter(acc_n, [ids], ones16)
        return carry

    lax.fori_loop(0, nvec, step, 0)

    pltpu.sync_copy(acc_c, shared.at[0, s, 0])
    pltpu.sync_copy(acc_d, shared.at[1, s, 0])
    pltpu.sync_copy(acc_n, shared.at[2, s, 0])
    plsc.subcore_barrier()

    accs = [[zeros16 for _ in range(BINS_PER_W // 16)] for _ in range(3)]
    for v in range(NS):
        b = v % 2
        for a in range(3):
            pltpu.sync_copy(
                shared.at[a, v, 0, pl.ds(s * BINS_PER_W, BINS_PER_W)],
                rbuf.at[a, b, 0])
        for a in range(3):
            for h in range(BINS_PER_W // 16):
                accs[a][h] = accs[a][h] + rbuf[a, b, 0, pl.ds(h * 16, 16)]
    for a, res in ((0, res0), (1, res1), (2, res2)):
        for h in range(BINS_PER_W // 16):
            res[pl.ds(h * 16, 16)] = accs[a][h]
    for a, res in ((0, res0), (1, res1), (2, res2)):
        pltpu.sync_copy(
            res,
            out_hbm.at[pl.ds(c * (3 * NUM_SEG) + a * NUM_SEG + s * BINS_PER_W,
                             BINS_PER_W)])


@jax.jit
def _sc_call(t_flat, ids32, xp_flat, x_flat, oh_flat, p0_flat, params):
    mesh = plsc.VectorSubcoreMesh(core_axis_name="c", subcore_axis_name="s")
    return pl.kernel(
        _sc_body,
        out_type=jax.ShapeDtypeStruct((NC * 3 * NUM_SEG,), jnp.float32),
        mesh=mesh,
        compiler_params=pltpu.CompilerParams(needs_layout_passes=False),
        scratch_types=[
            pltpu.VMEM((CHUNK,), jnp.float32),
            pltpu.VMEM((CHUNK,), jnp.int32),
            pltpu.VMEM((CHUNK * D,), jnp.float32),
            pltpu.VMEM((CHUNK * D,), jnp.float32),
            pltpu.VMEM((CHUNK * KDIM,), jnp.float32),
            pltpu.VMEM((CHUNK * KDIM,), jnp.float32),
            pltpu.VMEM((48,), jnp.float32),
            pltpu.VMEM((NUM_SEG,), jnp.float32),
            pltpu.VMEM((NUM_SEG,), jnp.float32),
            pltpu.VMEM((NUM_SEG,), jnp.float32),
            pltpu.VMEM((BINS_PER_W,), jnp.float32),
            pltpu.VMEM((BINS_PER_W,), jnp.float32),
            pltpu.VMEM((BINS_PER_W,), jnp.float32),
            pltpu.VMEM_SHARED((3, NS, 1, NUM_SEG), jnp.float32),
            pltpu.VMEM((3, 2, 1, BINS_PER_W), jnp.float32),
        ],
    )(t_flat, ids32, xp_flat, x_flat, oh_flat, p0_flat, params)


def _tc_epilogue_body(p_ref, o_ref):
    p = p_ref[...]
    srow = p[0] + p[1]
    cnt = jnp.maximum(srow[2], 1.0)
    o_ref[0, :] = srow[0] / cnt
    o_ref[1, :] = srow[1] / cnt


@jax.jit
def _tc_epilogue(partial):
    return pl.pallas_call(
        _tc_epilogue_body,
        out_shape=jax.ShapeDtypeStruct((2, NUM_SEG), jnp.float32),
    )(partial)


def kernel(t, sigma1, x_pred, x, segment_ids, beta1, one_hot_x, p_0, K):
    ln_s = jnp.log(sigma1[0])
    a = -2.0 * ln_s
    c1 = -ln_s
    c2 = K * beta1[0]
    params = jnp.concatenate([
        jnp.full((16,), a, jnp.float32),
        jnp.full((16,), c1, jnp.float32),
        jnp.full((16,), c2, jnp.float32),
    ])
    partial = _sc_call(
        t.reshape(-1),
        segment_ids.astype(jnp.int32),
        x_pred.reshape(-1),
        x.reshape(-1),
        one_hot_x.reshape(-1),
        p_0.reshape(-1),
        params,
    )
    return _tc_epilogue(partial.reshape(NC, 3, NUM_SEG))
